# baseline (speedup 1.0000x reference)
import jax
import jax.numpy as jnp
from jax import lax
from jax.experimental import pallas as pl
from jax.experimental.pallas import tpu as pltpu
from jax.experimental.pallas import tpu_sc as plsc

_N = 32640
_D = 512
_B = 256
_NS = 16
_RT = 2048
_C = 32
_NCH = _RT // _C
_HD = _D // 2
_G = 16


def _compute_chunk(buf, idb, acc, colofs):
    for g in range(_C // _G):
        idg = idb[0, pl.ds(_G * g, _G)]
        for rr in range(_G):
            spl = idg.at[jnp.full((_G,), rr, jnp.int32)].get(
                mode="promise_in_bounds")
            base = spl * _HD
            for j in range(_HD // 16):
                plsc.addupdate_scatter(
                    acc, [base + colofs[j]],
                    buf[_G * g + rr, pl.ds(16 * j, 16)])


def _sc_body(h_ref, ids_ref, part_ref, buf0, buf1, idb0, idb1, acc,
             sem0, sem1, isem0, isem1):
    c = lax.axis_index("c")
    s = lax.axis_index("s")
    row0 = s * _RT
    col0 = c * _HD
    nch = _NCH - 4 * (s // (_NS - 1))

    colofs = [jnp.arange(16, dtype=jnp.int32) + 16 * j
              for j in range(_HD // 16)]

    z = jnp.zeros((16,), jnp.float32)

    def _zrow(r, carry):
        base = jnp.full((16,), r, jnp.int32) * _HD
        for j in range(_HD // 16):
            plsc.store_scatter(acc, [base + colofs[j]], z)
        return carry

    lax.fori_loop(0, _B, _zrow, 0)

    bufs = (buf0, buf1)
    idbs = (idb0, idb1)
    sems = (sem0, sem1)
    isems = (isem0, isem1)

    def _issue(ch, par):
        pltpu.async_copy(
            h_ref.at[pl.ds(row0 + ch * _C, _C), pl.ds(col0, _HD)],
            bufs[par], sems[par])
        pltpu.async_copy(ids_ref.at[s, ch], idbs[par], isems[par])

    def _wait(par):
        pltpu.make_async_copy(
            h_ref.at[pl.ds(row0, _C), pl.ds(col0, _HD)],
            bufs[par], sems[par]).wait()
        pltpu.make_async_copy(ids_ref.at[s, 0], idbs[par], isems[par]).wait()

    _issue(0, 0)
    _issue(1, 1)

    def _pair(k, carry):
        ch = 2 * k
        _wait(0)
        _compute_chunk(buf0, idb0, acc, colofs)

        @pl.when(ch + 2 < nch)
        def _():
            _issue(ch + 2, 0)

        _wait(1)
        _compute_chunk(buf1, idb1, acc, colofs)

        @pl.when(ch + 3 < nch)
        def _():
            _issue(ch + 3, 1)

        return carry

    lax.fori_loop(0, nch // 2, _pair, 0)

    pltpu.sync_copy(acc, part_ref.at[s, c])


def _tc_sum_body(p_ref, out_ref):
    i = pl.program_id(0)

    @pl.when(i == 0)
    def _():
        out_ref[...] = jnp.zeros_like(out_ref)

    p = p_ref[0]
    out_ref[...] += jnp.concatenate([p[0], p[1]], axis=1)


def kernel(H_v, sizes):
    seg_ids = jnp.repeat(jnp.arange(_B, dtype=jnp.int32), sizes,
                         total_repeat_length=_N)
    seg_ids = jnp.concatenate(
        [seg_ids, jnp.zeros((_NS * _RT - _N,), jnp.int32)])
    ids4d = seg_ids.reshape(_NS, _NCH, 1, _C)
    mesh = plsc.VectorSubcoreMesh(core_axis_name="c", subcore_axis_name="s")
    sc = pl.kernel(
        _sc_body,
        out_type=jax.ShapeDtypeStruct((_NS, 2, _B * _HD), jnp.float32),
        mesh=mesh,
        compiler_params=pltpu.CompilerParams(use_tc_tiling_on_sc=False,
                                             needs_layout_passes=False),
        scratch_types=[
            pltpu.VMEM((_C, _HD), jnp.float32),
            pltpu.VMEM((_C, _HD), jnp.float32),
            pltpu.VMEM((1, _C), jnp.int32),
            pltpu.VMEM((1, _C), jnp.int32),
            pltpu.VMEM((_B * _HD,), jnp.float32),
            pltpu.SemaphoreType.DMA,
            pltpu.SemaphoreType.DMA,
            pltpu.SemaphoreType.DMA,
            pltpu.SemaphoreType.DMA,
        ],
    )
    partials = sc(H_v, ids4d).reshape(_NS, 2, _B, _HD)
    return pl.pallas_call(
        _tc_sum_body,
        grid=(_NS,),
        in_specs=[pl.BlockSpec((1, 2, _B, _HD), lambda i: (i, 0, 0, 0))],
        out_specs=pl.BlockSpec((_B, _D), lambda i: (0, 0)),
        out_shape=jax.ShapeDtypeStruct((_B, _D), jnp.float32),
    )(partials)
